# baseline (speedup 1.0000x reference)
import functools

import jax
import jax.numpy as jnp
from jax import lax
from jax.experimental import pallas as pl
from jax.experimental.pallas import tpu as pltpu
from jax.experimental.pallas import tpu_sc as plsc

NC = 2
NS = 16
LN = 16

FH = 128


def _sc_mesh():
    return plsc.VectorSubcoreMesh(
        core_axis_name="c", subcore_axis_name="s", num_cores=NC, num_subcores=NS
    )


@functools.partial(jax.jit, static_argnums=(1, 2, 3))
def _sc_degrees(dstp, ROWS, K, N_pad):
    STRIPE = N_pad // NS

    @functools.partial(
        pl.kernel,
        out_type=jax.ShapeDtypeStruct((NC, N_pad), jnp.float32),
        mesh=_sc_mesh(),
        scratch_types=[
            pltpu.VMEM((ROWS, K), jnp.int32),
            pltpu.VMEM((K,), jnp.float32),
            pltpu.VMEM((STRIPE,), jnp.float32),
            pltpu.VMEM_SHARED((N_pad,), jnp.float32),
        ],
    )
    def k(dst_hbm, out_hbm, dst_v, ones_v, zb_v, deg_sh):
        c = lax.axis_index("c")
        s = lax.axis_index("s")
        t = c * NS + s
        zero16 = jnp.zeros((LN,), jnp.float32)
        one16 = jnp.ones((LN,), jnp.float32)

        def z1(i, carry):
            zb_v[pl.ds(i * LN, LN)] = zero16
            return carry

        lax.fori_loop(0, STRIPE // LN, z1, 0)

        def o1(i, carry):
            ones_v[pl.ds(i * LN, LN)] = one16
            return carry

        lax.fori_loop(0, K // LN, o1, 0)

        pltpu.sync_copy(zb_v, deg_sh.at[pl.ds(s * STRIPE, STRIPE)])
        pltpu.sync_copy(dst_hbm.at[pl.ds(t * ROWS, ROWS)], dst_v)
        plsc.subcore_barrier()

        def cb(j, carry):
            pltpu.sync_copy(ones_v, deg_sh.at[dst_v.at[j]], add=True)
            return carry

        lax.fori_loop(0, ROWS, cb, 0)

        plsc.subcore_barrier()
        pltpu.sync_copy(deg_sh.at[pl.ds(s * STRIPE, STRIPE)], zb_v)
        pltpu.sync_copy(zb_v, out_hbm.at[c, pl.ds(s * STRIPE, STRIPE)])

    return k(dstp)


NBUF = 2


SPLIT = 2


@functools.partial(jax.jit, static_argnums=(4, 5, 6))
def _sc_aggregate(u0, u1, src2, dst2, R, N_pad, K):
    CH = R // NS
    STRIPE = N_pad // NS
    OUT_CH = STRIPE // K
    KS = K // SPLIT
    assert CH % NBUF == 0 and OUT_CH >= NBUF and KS % 8 == 0

    @functools.partial(
        pl.kernel,
        out_type=(
            jax.ShapeDtypeStruct((N_pad, FH), jnp.float32),
            jax.ShapeDtypeStruct((N_pad, FH), jnp.float32),
        ),
        mesh=_sc_mesh(),
        scratch_types=[
            pltpu.VMEM((CH // 2, K), jnp.int32),
            pltpu.VMEM((CH // 2, K), jnp.int32),
            pltpu.VMEM((K, FH), jnp.float32),
            pltpu.VMEM((K, FH), jnp.float32),
            pltpu.VMEM_SHARED((N_pad, FH), jnp.float32),
            pltpu.SemaphoreType.DMA,
            pltpu.SemaphoreType.DMA,
            pltpu.SemaphoreType.DMA,
            pltpu.SemaphoreType.DMA,
        ],
    )
    def k(u0_hbm, u1_hbm, src_hbm, dst_hbm, a0_hbm, a1_hbm,
          src_v, dst_v, buf0_v, buf1_v, acc_sh, gsem0, gsem1, ssem0, ssem1):
        c = lax.axis_index("c")
        s = lax.axis_index("s")
        bufs = [buf0_v, buf1_v]
        gsems = [gsem0, gsem1]
        ssems = [ssem0, ssem1]
        HC = CH // 2

        zero16 = jnp.zeros((LN,), jnp.float32)

        def zf(i, carry):
            r = i // (FH // LN)
            q = i % (FH // LN)
            buf0_v[r, pl.ds(q * LN, LN)] = zero16
            return carry

        lax.fori_loop(0, K * FH // LN, zf, 0)
        for j in range(OUT_CH):
            pltpu.sync_copy(buf0_v, acc_sh.at[pl.ds(s * STRIPE + j * K, K)])

        def run_half(u_hbm, out_hbm):
            def gstart(t, b):
                for h in range(SPLIT):
                    pltpu.async_copy(
                        u_hbm.at[src_v.at[t, pl.ds(h * KS, KS)]],
                        bufs[b].at[pl.ds(h * KS, KS)], gsems[b])

            def gwait(t, b):
                for h in range(SPLIT):
                    pltpu.make_async_copy(
                        u_hbm.at[src_v.at[t, pl.ds(h * KS, KS)]],
                        bufs[b].at[pl.ds(h * KS, KS)], gsems[b]).wait()

            def sstart(t, b):
                pltpu.async_copy(
                    bufs[b], acc_sh.at[dst_v.at[t]], ssems[b], add=True)

            def swait(t, b):
                pltpu.make_async_copy(
                    bufs[b], acc_sh.at[dst_v.at[t]], ssems[b]).wait()

            for p in range(2):
                pltpu.sync_copy(
                    src_hbm.at[pl.ds(s * CH + p * HC, HC)], src_v)
                pltpu.sync_copy(
                    dst_hbm.at[pl.ds(s * CH + p * HC, HC)], dst_v)
                for b in range(NBUF):
                    gstart(b, b)
                if p == 0:
                    plsc.subcore_barrier()

                def cb(g, carry):
                    for b in range(NBUF):
                        gwait(g + b, b)
                        sstart(g + b, b)
                    for b in range(NBUF):
                        t = g + b
                        swait(t, b)

                        @pl.when(t + NBUF < HC)
                        def _():
                            gstart(t + NBUF, b)

                    return carry

                lax.fori_loop(0, HC // NBUF, lambda i, cc: cb(i * NBUF, cc), 0)
            plsc.subcore_barrier()

            for j in range(OUT_CH):
                b = j % NBUF
                if j >= NBUF:
                    poff = s * STRIPE + (j - NBUF) * K
                    pltpu.make_async_copy(
                        bufs[b], out_hbm.at[pl.ds(poff, K)], gsems[b]).wait()
                off = s * STRIPE + j * K
                pltpu.sync_copy(acc_sh.at[pl.ds(off, K)], bufs[b])
                pltpu.async_copy(bufs[b], out_hbm.at[pl.ds(off, K)], gsems[b])
            for j in range(OUT_CH - NBUF, OUT_CH):
                b = j % NBUF
                off = s * STRIPE + j * K
                pltpu.make_async_copy(
                    bufs[b], out_hbm.at[pl.ds(off, K)], gsems[b]).wait()

        @pl.when(c == 0)
        def _():
            run_half(u0_hbm, a0_hbm)

        @pl.when(c == 1)
        def _():
            run_half(u1_hbm, a1_hbm)

    return k(u0, u1, src2, dst2)


def _mm1_body(x_ref, w_ref, d0_ref, d1_ref, out_ref):
    inv = lax.rsqrt(d0_ref[...] + d1_ref[...] + 1.0)
    acc = jnp.dot(x_ref[...], w_ref[...], preferred_element_type=jnp.float32)
    out_ref[...] = (acc * inv)[None]


def _tc_mm1(x, W, d0, d1, N, H, RB):
    ng = N // RB
    return pl.pallas_call(
        _mm1_body,
        grid=(ng, 2),
        in_specs=[
            pl.BlockSpec((RB, H), lambda i, j: (i, 0)),
            pl.BlockSpec((H, FH), lambda i, j: (0, j)),
            pl.BlockSpec((RB, 1), lambda i, j: (i, 0)),
            pl.BlockSpec((RB, 1), lambda i, j: (i, 0)),
        ],
        out_specs=pl.BlockSpec((1, RB, FH), lambda i, j: (j, i, 0)),
        out_shape=jax.ShapeDtypeStruct((2, N, FH), jnp.float32),
    )(x, W, d0, d1)


def _layer_body(a0_ref, a1_ref, u0_ref, u1_ref, d0_ref, d1_ref, b_ref, w_ref,
                out_ref):
    inv = lax.rsqrt(d0_ref[...] + d1_ref[...] + 1.0)
    b = b_ref[...]
    z0 = jnp.maximum(inv * (a0_ref[...] + u0_ref[...]) + b[:, :FH], 0.0)
    z1 = jnp.maximum(inv * (a1_ref[...] + u1_ref[...]) + b[:, FH:], 0.0)
    z = jnp.concatenate([z0, z1], axis=1)
    acc = jnp.dot(z, w_ref[...], preferred_element_type=jnp.float32)
    out_ref[...] = (acc * inv)[None]


def _tc_layer(a0, a1, u0, u1, d0, d1, b2, W, N, H, RB):
    ng = N // RB
    return pl.pallas_call(
        _layer_body,
        grid=(ng, 2),
        in_specs=[
            pl.BlockSpec((RB, FH), lambda i, j: (i, 0)),
            pl.BlockSpec((RB, FH), lambda i, j: (i, 0)),
            pl.BlockSpec((RB, FH), lambda i, j: (i, 0)),
            pl.BlockSpec((RB, FH), lambda i, j: (i, 0)),
            pl.BlockSpec((RB, 1), lambda i, j: (i, 0)),
            pl.BlockSpec((RB, 1), lambda i, j: (i, 0)),
            pl.BlockSpec((1, H), lambda i, j: (0, 0)),
            pl.BlockSpec((H, FH), lambda i, j: (0, j)),
        ],
        out_specs=pl.BlockSpec((1, RB, FH), lambda i, j: (j, i, 0)),
        out_shape=jax.ShapeDtypeStruct((2, N, FH), jnp.float32),
    )(a0, a1, u0, u1, d0, d1, b2, W)


def _final_body(a0_ref, a1_ref, u0_ref, u1_ref, d0_ref, d1_ref, b_ref,
                batch_ref, wl_ref, bl_ref, out_ref, psum, cnt, *, G, RB, ng):
    i = pl.program_id(0)

    @pl.when(i == 0)
    def _():
        psum[...] = jnp.zeros_like(psum)
        cnt[...] = jnp.zeros_like(cnt)

    inv = lax.rsqrt(d0_ref[...] + d1_ref[...] + 1.0)
    b = b_ref[...]
    z0 = jnp.maximum(inv * (a0_ref[...] + u0_ref[...]) + b[:, :FH], 0.0)
    z1 = jnp.maximum(inv * (a1_ref[...] + u1_ref[...]) + b[:, FH:], 0.0)
    z = jnp.concatenate([z0, z1], axis=1)

    bb = batch_ref[...]
    gids = lax.broadcasted_iota(jnp.int32, (RB, G), 1)
    P = (gids == bb).astype(jnp.float32)
    psum[...] += lax.dot_general(
        P, z, (((0,), (0,)), ((), ())),
        preferred_element_type=jnp.float32)
    csum = lax.dot_general(
        P, jnp.ones((RB, 1), jnp.float32), (((0,), (0,)), ((), ())),
        preferred_element_type=jnp.float32)
    cnt[...] += jnp.broadcast_to(csum, cnt.shape)

    @pl.when(i == ng - 1)
    def _():
        c = jnp.maximum(cnt[...], 1.0)
        pooled = psum[...] / jnp.concatenate([c, c], axis=1)
        out_ref[...] = (
            jnp.dot(pooled, wl_ref[...], preferred_element_type=jnp.float32)
            + bl_ref[...]
        )


def _tc_final(a0, a1, u0, u1, d0, d1, b2, batch2, Wl, bl2, N, H, G, C, RB):
    ng = N // RB
    return pl.pallas_call(
        functools.partial(_final_body, G=G, RB=RB, ng=ng),
        grid=(ng,),
        in_specs=[
            pl.BlockSpec((RB, FH), lambda i: (i, 0)),
            pl.BlockSpec((RB, FH), lambda i: (i, 0)),
            pl.BlockSpec((RB, FH), lambda i: (i, 0)),
            pl.BlockSpec((RB, FH), lambda i: (i, 0)),
            pl.BlockSpec((RB, 1), lambda i: (i, 0)),
            pl.BlockSpec((RB, 1), lambda i: (i, 0)),
            pl.BlockSpec((1, H), lambda i: (0, 0)),
            pl.BlockSpec((RB, 1), lambda i: (i, 0)),
            pl.BlockSpec((H, C), lambda i: (0, 0)),
            pl.BlockSpec((1, C), lambda i: (0, 0)),
        ],
        out_specs=pl.BlockSpec((G, C), lambda i: (0, 0)),
        out_shape=jax.ShapeDtypeStruct((G, C), jnp.float32),
        scratch_shapes=[
            pltpu.VMEM((G, 2 * FH), jnp.float32),
            pltpu.VMEM((G, FH), jnp.float32),
        ],
    )(a0, a1, u0, u1, d0, d1, b2, batch2, Wl, bl2)


def kernel(x, edge_index, batch, dropout, W1, b1, Wh0, bh0, Wh1, bh1, Wl, bl):
    N, D = x.shape
    H = W1.shape[1]
    C = Wl.shape[1]
    E = edge_index.shape[1]
    G = 64
    K = 80
    RB = 1000
    N_pad = ((N + NS * K - 1) // (NS * K)) * (NS * K)

    src = edge_index[0].astype(jnp.int32)
    dst = edge_index[1].astype(jnp.int32)

    R = ((-(-E // K) + NC * NS * 8 - 1) // (NC * NS * 8)) * (NC * NS * 8)
    EPAD = R * K
    srcp = jnp.concatenate(
        [src, jnp.zeros((EPAD - E,), jnp.int32)]).reshape(R, K)
    dstp = jnp.concatenate(
        [dst, jnp.full((EPAD - E,), N, jnp.int32)]).reshape(R, K)
    ROWS = R // (NC * NS)

    degp = _sc_degrees(dstp, ROWS, K, N_pad)
    d0 = degp[0, :N].reshape(N, 1)
    d1 = degp[1, :N].reshape(N, 1)

    batch2 = batch.astype(jnp.int32).reshape(N, 1)
    b1_2 = b1.reshape(1, H)
    bh0_2 = bh0.reshape(1, H)
    bh1_2 = bh1.reshape(1, H)
    bl_2 = bl.reshape(1, C)

    uu = _tc_mm1(x, W1, d0, d1, N, H, RB)
    u0, u1 = uu[0], uu[1]

    a0, a1 = _sc_aggregate(u0, u1, srcp, dstp, R, N_pad, K)
    uu = _tc_layer(a0[:N], a1[:N], u0, u1, d0, d1, b1_2, Wh0, N, H, RB)
    u0, u1 = uu[0], uu[1]

    a0, a1 = _sc_aggregate(u0, u1, srcp, dstp, R, N_pad, K)
    uu = _tc_layer(a0[:N], a1[:N], u0, u1, d0, d1, bh0_2, Wh1, N, H, RB)
    u0, u1 = uu[0], uu[1]

    a0, a1 = _sc_aggregate(u0, u1, srcp, dstp, R, N_pad, K)
    out = _tc_final(a0[:N], a1[:N], u0, u1, d0, d1, bh1_2, batch2, Wl, bl_2,
                    N, H, G, C, RB)
    return out


# speedup vs baseline: 1.0800x; 1.0800x over previous
import functools

import jax
import jax.numpy as jnp
from jax import lax
from jax.experimental import pallas as pl
from jax.experimental.pallas import tpu as pltpu
from jax.experimental.pallas import tpu_sc as plsc

NC = 2
NS = 16
LN = 16

FH = 128


def _sc_mesh():
    return plsc.VectorSubcoreMesh(
        core_axis_name="c", subcore_axis_name="s", num_cores=NC, num_subcores=NS
    )


@functools.partial(jax.jit, static_argnums=(1, 2, 3))
def _sc_degrees(dstp, ROWS, K, N_pad):
    STRIPE = N_pad // NS

    @functools.partial(
        pl.kernel,
        out_type=jax.ShapeDtypeStruct((NC, N_pad), jnp.float32),
        mesh=_sc_mesh(),
        scratch_types=[
            pltpu.VMEM((ROWS, K), jnp.int32),
            pltpu.VMEM((K,), jnp.float32),
            pltpu.VMEM((STRIPE,), jnp.float32),
            pltpu.VMEM_SHARED((N_pad,), jnp.float32),
        ],
    )
    def k(dst_hbm, out_hbm, dst_v, ones_v, zb_v, deg_sh):
        c = lax.axis_index("c")
        s = lax.axis_index("s")
        t = c * NS + s
        zero16 = jnp.zeros((LN,), jnp.float32)
        one16 = jnp.ones((LN,), jnp.float32)

        def z1(i, carry):
            zb_v[pl.ds(i * LN, LN)] = zero16
            return carry

        lax.fori_loop(0, STRIPE // LN, z1, 0)

        def o1(i, carry):
            ones_v[pl.ds(i * LN, LN)] = one16
            return carry

        lax.fori_loop(0, K // LN, o1, 0)

        pltpu.sync_copy(zb_v, deg_sh.at[pl.ds(s * STRIPE, STRIPE)])
        pltpu.sync_copy(dst_hbm.at[pl.ds(t * ROWS, ROWS)], dst_v)
        plsc.subcore_barrier()

        def cb(j, carry):
            pltpu.sync_copy(ones_v, deg_sh.at[dst_v.at[j]], add=True)
            return carry

        lax.fori_loop(0, ROWS, cb, 0)

        plsc.subcore_barrier()
        pltpu.sync_copy(deg_sh.at[pl.ds(s * STRIPE, STRIPE)], zb_v)
        pltpu.sync_copy(zb_v, out_hbm.at[c, pl.ds(s * STRIPE, STRIPE)])

    return k(dstp)


NBUF = 2
SPLIT = 2


@functools.partial(jax.jit, static_argnums=(4, 5, 6))
def _sc_aggregate(u0, u1, src2, dst2, R, N_pad, K):
    CH = R // NS
    STRIPE = N_pad // NS
    OUT_CH = STRIPE // K
    KS = K // SPLIT
    assert CH % (2 * NBUF) == 0 and OUT_CH >= NBUF and KS % 8 == 0

    @functools.partial(
        pl.kernel,
        out_type=(
            jax.ShapeDtypeStruct((N_pad, FH), jnp.float32),
            jax.ShapeDtypeStruct((N_pad, FH), jnp.float32),
        ),
        mesh=_sc_mesh(),
        scratch_types=[
            pltpu.VMEM((CH // 2, K), jnp.int32),
            pltpu.VMEM((CH // 2, K), jnp.int32),
            pltpu.VMEM((K, FH), jnp.float32),
            pltpu.VMEM((K, FH), jnp.float32),
            pltpu.VMEM_SHARED((N_pad, FH), jnp.float32),
            pltpu.SemaphoreType.DMA,
            pltpu.SemaphoreType.DMA,
        ],
    )
    def k(u0_hbm, u1_hbm, src_hbm, dst_hbm, a0_hbm, a1_hbm,
          src_v, dst_v, buf0_v, buf1_v, acc_sh, sem0, sem1):
        c = lax.axis_index("c")
        s = lax.axis_index("s")
        bufs = [buf0_v, buf1_v]
        gsems = [sem0, sem1]
        HC = CH // 2

        zero16 = jnp.zeros((LN,), jnp.float32)

        def zf(i, carry):
            r = i // (FH // LN)
            q = i % (FH // LN)
            buf0_v[r, pl.ds(q * LN, LN)] = zero16
            return carry

        lax.fori_loop(0, K * FH // LN, zf, 0)
        for j in range(OUT_CH):
            pltpu.sync_copy(buf0_v, acc_sh.at[pl.ds(s * STRIPE + j * K, K)])

        def run_half(u_hbm, out_hbm):
            def gstart(t, b):
                for h in range(SPLIT):
                    pltpu.async_copy(
                        u_hbm.at[src_v.at[t, pl.ds(h * KS, KS)]],
                        bufs[b].at[pl.ds(h * KS, KS)], gsems[b])

            def gwait(t, b):
                for h in range(SPLIT):
                    pltpu.make_async_copy(
                        u_hbm.at[src_v.at[t, pl.ds(h * KS, KS)]],
                        bufs[b].at[pl.ds(h * KS, KS)], gsems[b]).wait()

            def sdo(t, b):
                pltpu.sync_copy(bufs[b], acc_sh.at[dst_v.at[t]], add=True)

            for p in range(2):
                pltpu.sync_copy(
                    src_hbm.at[pl.ds(s * CH + p * HC, HC)], src_v)
                pltpu.sync_copy(
                    dst_hbm.at[pl.ds(s * CH + p * HC, HC)], dst_v)
                for b in range(NBUF):
                    gstart(b, b)
                if p == 0:
                    plsc.subcore_barrier()

                def cb(g, carry):
                    for b in range(NBUF):
                        t = g + b
                        gwait(t, b)
                        sdo(t, b)

                        @pl.when(t + NBUF < HC)
                        def _():
                            gstart(t + NBUF, b)

                    return carry

                lax.fori_loop(0, HC // NBUF, lambda i, cc: cb(i * NBUF, cc), 0)
            plsc.subcore_barrier()

            for j in range(OUT_CH):
                b = j % NBUF
                if j >= NBUF:
                    poff = s * STRIPE + (j - NBUF) * K
                    pltpu.make_async_copy(
                        bufs[b], out_hbm.at[pl.ds(poff, K)], gsems[b]).wait()
                off = s * STRIPE + j * K
                pltpu.sync_copy(acc_sh.at[pl.ds(off, K)], bufs[b])
                pltpu.async_copy(bufs[b], out_hbm.at[pl.ds(off, K)], gsems[b])
            for j in range(OUT_CH - NBUF, OUT_CH):
                b = j % NBUF
                off = s * STRIPE + j * K
                pltpu.make_async_copy(
                    bufs[b], out_hbm.at[pl.ds(off, K)], gsems[b]).wait()

        @pl.when(c == 0)
        def _():
            run_half(u0_hbm, a0_hbm)

        @pl.when(c == 1)
        def _():
            run_half(u1_hbm, a1_hbm)

    return k(u0, u1, src2, dst2)


def _mm1_body(x_ref, w_ref, d0_ref, d1_ref, out_ref):
    inv = lax.rsqrt(d0_ref[...] + d1_ref[...] + 1.0)
    acc = jnp.dot(x_ref[...], w_ref[...], preferred_element_type=jnp.float32)
    out_ref[...] = (acc * inv)[None]


def _tc_mm1(x, W, d0, d1, N, H, RB):
    ng = N // RB
    return pl.pallas_call(
        _mm1_body,
        grid=(ng, 2),
        in_specs=[
            pl.BlockSpec((RB, H), lambda i, j: (i, 0)),
            pl.BlockSpec((H, FH), lambda i, j: (0, j)),
            pl.BlockSpec((RB, 1), lambda i, j: (i, 0)),
            pl.BlockSpec((RB, 1), lambda i, j: (i, 0)),
        ],
        out_specs=pl.BlockSpec((1, RB, FH), lambda i, j: (j, i, 0)),
        out_shape=jax.ShapeDtypeStruct((2, N, FH), jnp.float32),
    )(x, W, d0, d1)


def _layer_body(a0_ref, a1_ref, u0_ref, u1_ref, d0_ref, d1_ref, b_ref, w_ref,
                out_ref):
    inv = lax.rsqrt(d0_ref[...] + d1_ref[...] + 1.0)
    b = b_ref[...]
    z0 = jnp.maximum(inv * (a0_ref[...] + u0_ref[...]) + b[:, :FH], 0.0)
    z1 = jnp.maximum(inv * (a1_ref[...] + u1_ref[...]) + b[:, FH:], 0.0)
    z = jnp.concatenate([z0, z1], axis=1)
    acc = jnp.dot(z, w_ref[...], preferred_element_type=jnp.float32)
    out_ref[...] = (acc * inv)[None]


def _tc_layer(a0, a1, u0, u1, d0, d1, b2, W, N, H, RB):
    ng = N // RB
    return pl.pallas_call(
        _layer_body,
        grid=(ng, 2),
        in_specs=[
            pl.BlockSpec((RB, FH), lambda i, j: (i, 0)),
            pl.BlockSpec((RB, FH), lambda i, j: (i, 0)),
            pl.BlockSpec((RB, FH), lambda i, j: (i, 0)),
            pl.BlockSpec((RB, FH), lambda i, j: (i, 0)),
            pl.BlockSpec((RB, 1), lambda i, j: (i, 0)),
            pl.BlockSpec((RB, 1), lambda i, j: (i, 0)),
            pl.BlockSpec((1, H), lambda i, j: (0, 0)),
            pl.BlockSpec((H, FH), lambda i, j: (0, j)),
        ],
        out_specs=pl.BlockSpec((1, RB, FH), lambda i, j: (j, i, 0)),
        out_shape=jax.ShapeDtypeStruct((2, N, FH), jnp.float32),
    )(a0, a1, u0, u1, d0, d1, b2, W)


def _final_body(a0_ref, a1_ref, u0_ref, u1_ref, d0_ref, d1_ref, b_ref,
                batch_ref, wl_ref, bl_ref, out_ref, psum, cnt, *, G, RB, ng):
    i = pl.program_id(0)

    @pl.when(i == 0)
    def _():
        psum[...] = jnp.zeros_like(psum)
        cnt[...] = jnp.zeros_like(cnt)

    inv = lax.rsqrt(d0_ref[...] + d1_ref[...] + 1.0)
    b = b_ref[...]
    z0 = jnp.maximum(inv * (a0_ref[...] + u0_ref[...]) + b[:, :FH], 0.0)
    z1 = jnp.maximum(inv * (a1_ref[...] + u1_ref[...]) + b[:, FH:], 0.0)
    z = jnp.concatenate([z0, z1], axis=1)

    bb = batch_ref[...]
    gids = lax.broadcasted_iota(jnp.int32, (RB, G), 1)
    P = (gids == bb).astype(jnp.float32)
    psum[...] += lax.dot_general(
        P, z, (((0,), (0,)), ((), ())),
        preferred_element_type=jnp.float32)
    csum = lax.dot_general(
        P, jnp.ones((RB, 1), jnp.float32), (((0,), (0,)), ((), ())),
        preferred_element_type=jnp.float32)
    cnt[...] += jnp.broadcast_to(csum, cnt.shape)

    @pl.when(i == ng - 1)
    def _():
        c = jnp.maximum(cnt[...], 1.0)
        pooled = psum[...] / jnp.concatenate([c, c], axis=1)
        out_ref[...] = (
            jnp.dot(pooled, wl_ref[...], preferred_element_type=jnp.float32)
            + bl_ref[...]
        )


def _tc_final(a0, a1, u0, u1, d0, d1, b2, batch2, Wl, bl2, N, H, G, C, RB):
    ng = N // RB
    return pl.pallas_call(
        functools.partial(_final_body, G=G, RB=RB, ng=ng),
        grid=(ng,),
        in_specs=[
            pl.BlockSpec((RB, FH), lambda i: (i, 0)),
            pl.BlockSpec((RB, FH), lambda i: (i, 0)),
            pl.BlockSpec((RB, FH), lambda i: (i, 0)),
            pl.BlockSpec((RB, FH), lambda i: (i, 0)),
            pl.BlockSpec((RB, 1), lambda i: (i, 0)),
            pl.BlockSpec((RB, 1), lambda i: (i, 0)),
            pl.BlockSpec((1, H), lambda i: (0, 0)),
            pl.BlockSpec((RB, 1), lambda i: (i, 0)),
            pl.BlockSpec((H, C), lambda i: (0, 0)),
            pl.BlockSpec((1, C), lambda i: (0, 0)),
        ],
        out_specs=pl.BlockSpec((G, C), lambda i: (0, 0)),
        out_shape=jax.ShapeDtypeStruct((G, C), jnp.float32),
        scratch_shapes=[
            pltpu.VMEM((G, 2 * FH), jnp.float32),
            pltpu.VMEM((G, FH), jnp.float32),
        ],
    )(a0, a1, u0, u1, d0, d1, b2, batch2, Wl, bl2)


def kernel(x, edge_index, batch, dropout, W1, b1, Wh0, bh0, Wh1, bh1, Wl, bl):
    N, D = x.shape
    H = W1.shape[1]
    C = Wl.shape[1]
    E = edge_index.shape[1]
    G = 64
    K = 80
    RB = 1000
    N_pad = ((N + NS * K - 1) // (NS * K)) * (NS * K)

    src = edge_index[0].astype(jnp.int32)
    dst = edge_index[1].astype(jnp.int32)

    R = ((-(-E // K) + NC * NS * 8 - 1) // (NC * NS * 8)) * (NC * NS * 8)
    EPAD = R * K
    srcp = jnp.concatenate(
        [src, jnp.zeros((EPAD - E,), jnp.int32)]).reshape(R, K)
    dstp = jnp.concatenate(
        [dst, jnp.full((EPAD - E,), N, jnp.int32)]).reshape(R, K)
    ROWS = R // (NC * NS)

    degp = _sc_degrees(dstp, ROWS, K, N_pad)
    d0 = degp[0, :N].reshape(N, 1)
    d1 = degp[1, :N].reshape(N, 1)

    batch2 = batch.astype(jnp.int32).reshape(N, 1)
    b1_2 = b1.reshape(1, H)
    bh0_2 = bh0.reshape(1, H)
    bh1_2 = bh1.reshape(1, H)
    bl_2 = bl.reshape(1, C)

    uu = _tc_mm1(x, W1, d0, d1, N, H, RB)
    u0, u1 = uu[0], uu[1]

    a0, a1 = _sc_aggregate(u0, u1, srcp, dstp, R, N_pad, K)
    uu = _tc_layer(a0[:N], a1[:N], u0, u1, d0, d1, b1_2, Wh0, N, H, RB)
    u0, u1 = uu[0], uu[1]

    a0, a1 = _sc_aggregate(u0, u1, srcp, dstp, R, N_pad, K)
    uu = _tc_layer(a0[:N], a1[:N], u0, u1, d0, d1, bh0_2, Wh1, N, H, RB)
    u0, u1 = uu[0], uu[1]

    a0, a1 = _sc_aggregate(u0, u1, srcp, dstp, R, N_pad, K)
    out = _tc_final(a0[:N], a1[:N], u0, u1, d0, d1, bh1_2, batch2, Wl, bl_2,
                    N, H, G, C, RB)
    return out


# speedup vs baseline: 1.1565x; 1.0709x over previous
import functools

import jax
import jax.numpy as jnp
from jax import lax
from jax.experimental import pallas as pl
from jax.experimental.pallas import tpu as pltpu
from jax.experimental.pallas import tpu_sc as plsc

NC = 2
NS = 16
LN = 16

FH = 128


def _sc_mesh():
    return plsc.VectorSubcoreMesh(
        core_axis_name="c", subcore_axis_name="s", num_cores=NC, num_subcores=NS
    )


@functools.partial(jax.jit, static_argnums=(1, 2, 3))
def _sc_degrees(dstp, ROWS, K, N_pad):
    STRIPE = N_pad // NS

    @functools.partial(
        pl.kernel,
        out_type=jax.ShapeDtypeStruct((NC, N_pad), jnp.float32),
        mesh=_sc_mesh(),
        scratch_types=[
            pltpu.VMEM((ROWS, K), jnp.int32),
            pltpu.VMEM((K,), jnp.float32),
            pltpu.VMEM((STRIPE,), jnp.float32),
            pltpu.VMEM_SHARED((N_pad,), jnp.float32),
        ],
    )
    def k(dst_hbm, out_hbm, dst_v, ones_v, zb_v, deg_sh):
        c = lax.axis_index("c")
        s = lax.axis_index("s")
        t = c * NS + s
        zero16 = jnp.zeros((LN,), jnp.float32)
        one16 = jnp.ones((LN,), jnp.float32)

        def z1(i, carry):
            zb_v[pl.ds(i * LN, LN)] = zero16
            return carry

        lax.fori_loop(0, STRIPE // LN, z1, 0)

        def o1(i, carry):
            ones_v[pl.ds(i * LN, LN)] = one16
            return carry

        lax.fori_loop(0, K // LN, o1, 0)

        pltpu.sync_copy(zb_v, deg_sh.at[pl.ds(s * STRIPE, STRIPE)])
        pltpu.sync_copy(dst_hbm.at[pl.ds(t * ROWS, ROWS)], dst_v)
        plsc.subcore_barrier()

        def cb(j, carry):
            pltpu.sync_copy(ones_v, deg_sh.at[dst_v.at[j]], add=True)
            return carry

        lax.fori_loop(0, ROWS, cb, 0)

        plsc.subcore_barrier()
        pltpu.sync_copy(deg_sh.at[pl.ds(s * STRIPE, STRIPE)], zb_v)
        pltpu.sync_copy(zb_v, out_hbm.at[c, pl.ds(s * STRIPE, STRIPE)])

    return k(dstp)


NBUF = 2
SPLIT = 2


@functools.partial(jax.jit, static_argnums=(4, 5, 6, 7))
def _sc_aggregate(u0, u1, src2, dst2, R, N_pad, K, NPASS):
    CH = R // NS
    STRIPE = N_pad // NS
    OUT_CH = STRIPE // K
    KS = K // SPLIT
    assert CH % (NPASS * NBUF) == 0 and OUT_CH >= NBUF and KS % 8 == 0

    @functools.partial(
        pl.kernel,
        out_type=(
            jax.ShapeDtypeStruct((N_pad, FH), jnp.float32),
            jax.ShapeDtypeStruct((N_pad, FH), jnp.float32),
        ),
        mesh=_sc_mesh(),
        scratch_types=[
            pltpu.VMEM((CH // NPASS, K), jnp.int32),
            pltpu.VMEM((CH // NPASS, K), jnp.int32),
            pltpu.VMEM((K, FH), jnp.float32),
            pltpu.VMEM((K, FH), jnp.float32),
            pltpu.VMEM_SHARED((N_pad, FH), jnp.float32),
            pltpu.SemaphoreType.DMA,
            pltpu.SemaphoreType.DMA,
        ],
    )
    def k(u0_hbm, u1_hbm, src_hbm, dst_hbm, a0_hbm, a1_hbm,
          src_v, dst_v, buf0_v, buf1_v, acc_sh, sem0, sem1):
        c = lax.axis_index("c")
        s = lax.axis_index("s")
        bufs = [buf0_v, buf1_v]
        gsems = [sem0, sem1]
        HC = CH // NPASS

        zero16 = jnp.zeros((LN,), jnp.float32)

        def zf(i, carry):
            r = i // (FH // LN)
            q = i % (FH // LN)
            buf0_v[r, pl.ds(q * LN, LN)] = zero16
            return carry

        lax.fori_loop(0, K * FH // LN, zf, 0)
        for j in range(OUT_CH):
            pltpu.sync_copy(buf0_v, acc_sh.at[pl.ds(s * STRIPE + j * K, K)])

        def run_half(u_hbm, out_hbm):
            def gstart(t, b):
                for h in range(SPLIT):
                    pltpu.async_copy(
                        u_hbm.at[src_v.at[t, pl.ds(h * KS, KS)]],
                        bufs[b].at[pl.ds(h * KS, KS)], gsems[b])

            def gwait(t, b):
                for h in range(SPLIT):
                    pltpu.make_async_copy(
                        u_hbm.at[src_v.at[t, pl.ds(h * KS, KS)]],
                        bufs[b].at[pl.ds(h * KS, KS)], gsems[b]).wait()

            def sdo(t, b):
                pltpu.sync_copy(bufs[b], acc_sh.at[dst_v.at[t]], add=True)

            for p in range(NPASS):
                pltpu.sync_copy(
                    src_hbm.at[pl.ds(s * CH + p * HC, HC)], src_v)
                pltpu.sync_copy(
                    dst_hbm.at[pl.ds(s * CH + p * HC, HC)], dst_v)
                for b in range(NBUF):
                    gstart(b, b)
                if p == 0:
                    plsc.subcore_barrier()

                def cb(g, carry):
                    for b in range(NBUF):
                        t = g + b
                        gwait(t, b)
                        sdo(t, b)

                        @pl.when(t + NBUF < HC)
                        def _():
                            gstart(t + NBUF, b)

                    return carry

                lax.fori_loop(0, HC // NBUF, lambda i, cc: cb(i * NBUF, cc), 0)
            plsc.subcore_barrier()

            for j in range(OUT_CH):
                b = j % NBUF
                if j >= NBUF:
                    poff = s * STRIPE + (j - NBUF) * K
                    pltpu.make_async_copy(
                        bufs[b], out_hbm.at[pl.ds(poff, K)], gsems[b]).wait()
                off = s * STRIPE + j * K
                pltpu.sync_copy(acc_sh.at[pl.ds(off, K)], bufs[b])
                pltpu.async_copy(bufs[b], out_hbm.at[pl.ds(off, K)], gsems[b])
            for j in range(OUT_CH - NBUF, OUT_CH):
                b = j % NBUF
                off = s * STRIPE + j * K
                pltpu.make_async_copy(
                    bufs[b], out_hbm.at[pl.ds(off, K)], gsems[b]).wait()

        @pl.when(c == 0)
        def _():
            run_half(u0_hbm, a0_hbm)

        @pl.when(c == 1)
        def _():
            run_half(u1_hbm, a1_hbm)

    return k(u0, u1, src2, dst2)


def _mm1_body(x_ref, w_ref, d0_ref, d1_ref, out_ref):
    inv = lax.rsqrt(d0_ref[...] + d1_ref[...] + 1.0)
    acc = jnp.dot(x_ref[...], w_ref[...], preferred_element_type=jnp.float32)
    out_ref[...] = (acc * inv)[None]


def _tc_mm1(x, W, d0, d1, N, H, RB):
    ng = N // RB
    return pl.pallas_call(
        _mm1_body,
        grid=(ng, 2),
        in_specs=[
            pl.BlockSpec((RB, H), lambda i, j: (i, 0)),
            pl.BlockSpec((H, FH), lambda i, j: (0, j)),
            pl.BlockSpec((RB, 1), lambda i, j: (i, 0)),
            pl.BlockSpec((RB, 1), lambda i, j: (i, 0)),
        ],
        out_specs=pl.BlockSpec((1, RB, FH), lambda i, j: (j, i, 0)),
        out_shape=jax.ShapeDtypeStruct((2, N, FH), jnp.float32),
    )(x, W, d0, d1)


def _layer_body(a0_ref, a1_ref, u0_ref, u1_ref, d0_ref, d1_ref, b_ref, w_ref,
                out_ref):
    inv = lax.rsqrt(d0_ref[...] + d1_ref[...] + 1.0)
    b = b_ref[...]
    z0 = jnp.maximum(inv * (a0_ref[...] + u0_ref[...]) + b[:, :FH], 0.0)
    z1 = jnp.maximum(inv * (a1_ref[...] + u1_ref[...]) + b[:, FH:], 0.0)
    z = jnp.concatenate([z0, z1], axis=1)
    acc = jnp.dot(z, w_ref[...], preferred_element_type=jnp.float32)
    out_ref[...] = (acc * inv)[None]


def _tc_layer(a0, a1, u0, u1, d0, d1, b2, W, N, H, RB):
    ng = N // RB
    return pl.pallas_call(
        _layer_body,
        grid=(ng, 2),
        in_specs=[
            pl.BlockSpec((RB, FH), lambda i, j: (i, 0)),
            pl.BlockSpec((RB, FH), lambda i, j: (i, 0)),
            pl.BlockSpec((RB, FH), lambda i, j: (i, 0)),
            pl.BlockSpec((RB, FH), lambda i, j: (i, 0)),
            pl.BlockSpec((RB, 1), lambda i, j: (i, 0)),
            pl.BlockSpec((RB, 1), lambda i, j: (i, 0)),
            pl.BlockSpec((1, H), lambda i, j: (0, 0)),
            pl.BlockSpec((H, FH), lambda i, j: (0, j)),
        ],
        out_specs=pl.BlockSpec((1, RB, FH), lambda i, j: (j, i, 0)),
        out_shape=jax.ShapeDtypeStruct((2, N, FH), jnp.float32),
    )(a0, a1, u0, u1, d0, d1, b2, W)


def _final_body(a0_ref, a1_ref, u0_ref, u1_ref, d0_ref, d1_ref, b_ref,
                batch_ref, wl_ref, bl_ref, out_ref, psum, cnt, *, G, RB, ng):
    i = pl.program_id(0)

    @pl.when(i == 0)
    def _():
        psum[...] = jnp.zeros_like(psum)
        cnt[...] = jnp.zeros_like(cnt)

    inv = lax.rsqrt(d0_ref[...] + d1_ref[...] + 1.0)
    b = b_ref[...]
    z0 = jnp.maximum(inv * (a0_ref[...] + u0_ref[...]) + b[:, :FH], 0.0)
    z1 = jnp.maximum(inv * (a1_ref[...] + u1_ref[...]) + b[:, FH:], 0.0)
    z = jnp.concatenate([z0, z1], axis=1)

    bb = batch_ref[...]
    gids = lax.broadcasted_iota(jnp.int32, (RB, G), 1)
    P = (gids == bb).astype(jnp.float32)
    psum[...] += lax.dot_general(
        P, z, (((0,), (0,)), ((), ())),
        preferred_element_type=jnp.float32)
    csum = lax.dot_general(
        P, jnp.ones((RB, 1), jnp.float32), (((0,), (0,)), ((), ())),
        preferred_element_type=jnp.float32)
    cnt[...] += jnp.broadcast_to(csum, cnt.shape)

    @pl.when(i == ng - 1)
    def _():
        c = jnp.maximum(cnt[...], 1.0)
        pooled = psum[...] / jnp.concatenate([c, c], axis=1)
        out_ref[...] = (
            jnp.dot(pooled, wl_ref[...], preferred_element_type=jnp.float32)
            + bl_ref[...]
        )


def _tc_final(a0, a1, u0, u1, d0, d1, b2, batch2, Wl, bl2, N, H, G, C, RB):
    ng = N // RB
    return pl.pallas_call(
        functools.partial(_final_body, G=G, RB=RB, ng=ng),
        grid=(ng,),
        in_specs=[
            pl.BlockSpec((RB, FH), lambda i: (i, 0)),
            pl.BlockSpec((RB, FH), lambda i: (i, 0)),
            pl.BlockSpec((RB, FH), lambda i: (i, 0)),
            pl.BlockSpec((RB, FH), lambda i: (i, 0)),
            pl.BlockSpec((RB, 1), lambda i: (i, 0)),
            pl.BlockSpec((RB, 1), lambda i: (i, 0)),
            pl.BlockSpec((1, H), lambda i: (0, 0)),
            pl.BlockSpec((RB, 1), lambda i: (i, 0)),
            pl.BlockSpec((H, C), lambda i: (0, 0)),
            pl.BlockSpec((1, C), lambda i: (0, 0)),
        ],
        out_specs=pl.BlockSpec((G, C), lambda i: (0, 0)),
        out_shape=jax.ShapeDtypeStruct((G, C), jnp.float32),
        scratch_shapes=[
            pltpu.VMEM((G, 2 * FH), jnp.float32),
            pltpu.VMEM((G, FH), jnp.float32),
        ],
    )(a0, a1, u0, u1, d0, d1, b2, batch2, Wl, bl2)


def kernel(x, edge_index, batch, dropout, W1, b1, Wh0, bh0, Wh1, bh1, Wl, bl):
    N, D = x.shape
    H = W1.shape[1]
    C = Wl.shape[1]
    E = edge_index.shape[1]
    G = 64
    K = 80
    KA = 128
    NPASS = 10
    RB = 1000
    N_pad = ((N + NS * KA - 1) // (NS * KA)) * (NS * KA)

    src = edge_index[0].astype(jnp.int32)
    dst = edge_index[1].astype(jnp.int32)

    R = ((-(-E // K) + NC * NS * 8 - 1) // (NC * NS * 8)) * (NC * NS * 8)
    EPAD = R * K
    srcf = jnp.concatenate([src, jnp.zeros((EPAD - E,), jnp.int32)])
    dstf = jnp.concatenate([dst, jnp.full((EPAD - E,), N, jnp.int32)])
    dstp = dstf.reshape(R, K)
    RA = EPAD // KA
    srcA = srcf.reshape(RA, KA)
    dstA = dstf.reshape(RA, KA)
    ROWS = R // (NC * NS)

    degp = _sc_degrees(dstp, ROWS, K, N_pad)
    d0 = degp[0, :N].reshape(N, 1)
    d1 = degp[1, :N].reshape(N, 1)

    batch2 = batch.astype(jnp.int32).reshape(N, 1)
    b1_2 = b1.reshape(1, H)
    bh0_2 = bh0.reshape(1, H)
    bh1_2 = bh1.reshape(1, H)
    bl_2 = bl.reshape(1, C)

    uu = _tc_mm1(x, W1, d0, d1, N, H, RB)
    u0, u1 = uu[0], uu[1]

    a0, a1 = _sc_aggregate(u0, u1, srcA, dstA, RA, N_pad, KA, NPASS)
    uu = _tc_layer(a0[:N], a1[:N], u0, u1, d0, d1, b1_2, Wh0, N, H, RB)
    u0, u1 = uu[0], uu[1]

    a0, a1 = _sc_aggregate(u0, u1, srcA, dstA, RA, N_pad, KA, NPASS)
    uu = _tc_layer(a0[:N], a1[:N], u0, u1, d0, d1, bh0_2, Wh1, N, H, RB)
    u0, u1 = uu[0], uu[1]

    a0, a1 = _sc_aggregate(u0, u1, srcA, dstA, RA, N_pad, KA, NPASS)
    out = _tc_final(a0[:N], a1[:N], u0, u1, d0, d1, bh1_2, batch2, Wl, bl_2,
                    N, H, G, C, RB)
    return out
